# baseline (speedup 1.0000x reference)
import jax
import jax.numpy as jnp
from jax import lax
from jax.experimental import pallas as pl
from jax.experimental.pallas import tpu as pltpu
from jax.experimental.pallas import tpu_sc as plsc

NC = 2
NS = 16
NW = NC * NS
L = 16

B, P, CHN = 128, 20000, 2

P_SC = 12800
P_TC = P - P_SC
POS_W = P_SC // NW
CHUNK_P = 25
NCHUNK = POS_W // CHUNK_P
MSK_C = CHUNK_P * B
VAL_C = CHUNK_P * B * CHN

PB = 400
NTCB = P_TC // PB
TCOFF = P_SC // PB


def _partial_body(regr_hbm, gt_hbm, mask_hbm, part_hbm,
                  mk0, mk1, rr0, rr1, gg0, gg1, stage_v,
                  sm0, sm1, sr0, sr1, sg0, sg1):
    cid = lax.axis_index("c")
    sid = lax.axis_index("s")
    wid = cid * NS + sid

    mbase = wid * POS_W * B
    fbase = wid * POS_W * B * CHN

    mk = (mk0, mk1)
    rr = (rr0, rr1)
    gg = (gg0, gg1)
    sems = ((sm0, sr0, sg0), (sm1, sr1, sg1))

    def start(s, c):
        sm, sr, sg = sems[s]
        pltpu.async_copy(mask_hbm.at[pl.ds(mbase + c * MSK_C, MSK_C)], mk[s], sm)
        pltpu.async_copy(regr_hbm.at[pl.ds(fbase + c * VAL_C, VAL_C)], rr[s], sr)
        pltpu.async_copy(gt_hbm.at[pl.ds(fbase + c * VAL_C, VAL_C)], gg[s], sg)

    def wait(s):
        sm, sr, sg = sems[s]
        pltpu.make_async_copy(mask_hbm.at[pl.ds(0, MSK_C)], mk[s], sm).wait()
        pltpu.make_async_copy(regr_hbm.at[pl.ds(0, VAL_C)], rr[s], sr).wait()
        pltpu.make_async_copy(gt_hbm.at[pl.ds(0, VAL_C)], gg[s], sg).wait()

    def process(s, c, acc, cnt):
        wait(s)
        mk_s, rr_s, gg_s = mk[s], rr[s], gg[s]

        def pstep(i, carry, mk_s=mk_s, rr_s=rr_s, gg_s=gg_s):
            acc2, cnt2 = carry
            mo = i * (B // 2)
            fo = (i >> 1) * B * CHN + (i & 1) * (B // 2)
            for j in range(B // (2 * L)):
                m = mk_s[pl.ds(mo + j * L, L)].astype(jnp.float32)
                r0 = rr_s[pl.ds(fo + j * L, L)]
                g0 = gg_s[pl.ds(fo + j * L, L)]
                r1 = rr_s[pl.ds(fo + B + j * L, L)]
                g1 = gg_s[pl.ds(fo + B + j * L, L)]
                acc2 = acc2 + (jnp.abs(r0 - g0) + jnp.abs(r1 - g1)) * m
                cnt2 = cnt2 + m
            return acc2, cnt2

        acc, cnt = lax.fori_loop(0, 2 * CHUNK_P, pstep, (acc, cnt))

        @pl.when(c + 2 < NCHUNK)
        def _():
            start(s, c + 2)
        return acc, cnt

    zero = jnp.zeros((L,), jnp.float32)
    start(0, 0)
    start(1, 1)

    def pair_body(c2, carry):
        acc, cnt = carry
        acc, cnt = process(0, c2 * 2, acc, cnt)
        acc, cnt = process(1, c2 * 2 + 1, acc, cnt)
        return acc, cnt

    acc, cnt = lax.fori_loop(0, NCHUNK // 2, pair_body, (zero, zero))
    if NCHUNK % 2:
        acc, cnt = process(0, NCHUNK - 1, acc, cnt)

    stage_v[pl.ds(0, L)] = acc
    stage_v[pl.ds(L, L)] = cnt
    pltpu.sync_copy(stage_v, part_hbm.at[wid])


def _tc_body(regr_ref, gt_ref, mask_ref, loss_ref, cnt_ref):
    i = pl.program_id(0)
    m = mask_ref[...].astype(jnp.float32)
    d = jnp.abs(regr_ref[...] - gt_ref[...])
    me = jnp.broadcast_to(m[:, None, :], (PB, 2, B)).reshape(2 * PB, B)
    bl = jnp.sum(d * me, axis=0, keepdims=True)
    bc = jnp.sum(m, axis=0, keepdims=True)

    @pl.when(i == 0)
    def _():
        loss_ref[...] = jnp.zeros((1, B), jnp.float32)
        cnt_ref[...] = jnp.zeros((1, B), jnp.float32)

    loss_ref[...] += bl
    cnt_ref[...] += bc


def _combine_body(part_ref, tcl_ref, tcc_ref, out_ref):
    s_loss = jnp.sum(part_ref[:, :L]) + jnp.sum(tcl_ref[...])
    s_cnt = jnp.sum(part_ref[:, L:]) + jnp.sum(tcc_ref[...])
    num = s_cnt * 2.0
    out_ref[...] = jnp.full((1, L), s_loss / (num + 1e-4), jnp.float32)


@jax.jit
def _masked_l1(regr, gt_regr, mask):
    regr_f = jnp.transpose(regr, (1, 2, 0)).reshape(-1)
    gt_f = jnp.transpose(gt_regr, (1, 2, 0)).reshape(-1)
    mask_f = jnp.transpose(mask, (1, 0)).reshape(-1)
    regr_2 = regr_f.reshape(P * CHN, B)
    gt_2 = gt_f.reshape(P * CHN, B)
    mask_2 = mask_f.reshape(P, B)

    mesh = plsc.VectorSubcoreMesh(core_axis_name="c", subcore_axis_name="s")
    partials = pl.kernel(
        _partial_body,
        out_type=jax.ShapeDtypeStruct((NW, 2 * L), jnp.float32),
        mesh=mesh,
        compiler_params=pltpu.CompilerParams(needs_layout_passes=False),
        scratch_types=[
            pltpu.VMEM((MSK_C,), jnp.int32),
            pltpu.VMEM((MSK_C,), jnp.int32),
            pltpu.VMEM((VAL_C,), jnp.float32),
            pltpu.VMEM((VAL_C,), jnp.float32),
            pltpu.VMEM((VAL_C,), jnp.float32),
            pltpu.VMEM((VAL_C,), jnp.float32),
            pltpu.VMEM((2 * L,), jnp.float32),
            pltpu.SemaphoreType.DMA,
            pltpu.SemaphoreType.DMA,
            pltpu.SemaphoreType.DMA,
            pltpu.SemaphoreType.DMA,
            pltpu.SemaphoreType.DMA,
            pltpu.SemaphoreType.DMA,
        ],
    )(regr_f, gt_f, mask_f)

    tc_loss, tc_cnt = pl.pallas_call(
        _tc_body,
        grid=(NTCB,),
        in_specs=[
            pl.BlockSpec((2 * PB, B), lambda i: (TCOFF + i, 0)),
            pl.BlockSpec((2 * PB, B), lambda i: (TCOFF + i, 0)),
            pl.BlockSpec((PB, B), lambda i: (TCOFF + i, 0)),
        ],
        out_specs=[
            pl.BlockSpec((1, B), lambda i: (0, 0)),
            pl.BlockSpec((1, B), lambda i: (0, 0)),
        ],
        out_shape=[
            jax.ShapeDtypeStruct((1, B), jnp.float32),
            jax.ShapeDtypeStruct((1, B), jnp.float32),
        ],
    )(regr_2, gt_2, mask_2)

    out = pl.pallas_call(
        _combine_body,
        out_shape=jax.ShapeDtypeStruct((1, L), jnp.float32),
    )(partials, tc_loss, tc_cnt)
    return out


def kernel(regr, gt_regr, mask):
    out = _masked_l1(regr, gt_regr, mask)
    return out[0, 0]


# speedup vs baseline: 1.0012x; 1.0012x over previous
import jax
import jax.numpy as jnp
from jax import lax
from jax.experimental import pallas as pl
from jax.experimental.pallas import tpu as pltpu
from jax.experimental.pallas import tpu_sc as plsc

NC = 2
NS = 16
NW = NC * NS
L = 16

B, P, CHN = 128, 20000, 2

P_SC = 12800
P_TC = P - P_SC
POS_W = P_SC // NW
CHUNK_P = 25
NCHUNK = POS_W // CHUNK_P
MSK_C = CHUNK_P * B
VAL_C = CHUNK_P * B * CHN

PB = 400
NTCB = P_TC // PB
TCOFF = P_SC // PB


def _partial_body(regr_hbm, gt_hbm, mask_hbm, part_hbm,
                  mk0, mk1, rr0, rr1, gg0, gg1, stage_v,
                  sm0, sm1, sr0, sr1, sg0, sg1):
    cid = lax.axis_index("c")
    sid = lax.axis_index("s")
    wid = cid * NS + sid

    mbase = wid * POS_W * B
    fbase = wid * POS_W * B * CHN

    mk = (mk0, mk1)
    rr = (rr0, rr1)
    gg = (gg0, gg1)
    sems = ((sm0, sr0, sg0), (sm1, sr1, sg1))

    def start(s, c):
        sm, sr, sg = sems[s]
        pltpu.async_copy(mask_hbm.at[pl.ds(mbase + c * MSK_C, MSK_C)], mk[s], sm)
        pltpu.async_copy(regr_hbm.at[pl.ds(fbase + c * VAL_C, VAL_C)], rr[s], sr)
        pltpu.async_copy(gt_hbm.at[pl.ds(fbase + c * VAL_C, VAL_C)], gg[s], sg)

    def wait(s):
        sm, sr, sg = sems[s]
        pltpu.make_async_copy(mask_hbm.at[pl.ds(0, MSK_C)], mk[s], sm).wait()
        pltpu.make_async_copy(regr_hbm.at[pl.ds(0, VAL_C)], rr[s], sr).wait()
        pltpu.make_async_copy(gt_hbm.at[pl.ds(0, VAL_C)], gg[s], sg).wait()

    def process(s, c, acc, cnt):
        wait(s)
        mk_s, rr_s, gg_s = mk[s], rr[s], gg[s]

        def pstep(i, carry, mk_s=mk_s, rr_s=rr_s, gg_s=gg_s):
            acc2, cnt2 = carry
            mo = i * (B // 2)
            fo = (i >> 1) * B * CHN + (i & 1) * (B // 2)
            for j in range(B // (2 * L)):
                m = mk_s[pl.ds(mo + j * L, L)].astype(jnp.float32)
                r0 = rr_s[pl.ds(fo + j * L, L)]
                g0 = gg_s[pl.ds(fo + j * L, L)]
                r1 = rr_s[pl.ds(fo + B + j * L, L)]
                g1 = gg_s[pl.ds(fo + B + j * L, L)]
                acc2 = acc2 + (jnp.abs(r0 - g0) + jnp.abs(r1 - g1)) * m
                cnt2 = cnt2 + m
            return acc2, cnt2

        acc, cnt = lax.fori_loop(0, 2 * CHUNK_P, pstep, (acc, cnt))

        @pl.when(c + 2 < NCHUNK)
        def _():
            start(s, c + 2)
        return acc, cnt

    zero = jnp.zeros((L,), jnp.float32)
    start(0, 0)
    start(1, 1)

    def pair_body(c2, carry):
        acc, cnt = carry
        acc, cnt = process(0, c2 * 2, acc, cnt)
        acc, cnt = process(1, c2 * 2 + 1, acc, cnt)
        return acc, cnt

    acc, cnt = lax.fori_loop(0, NCHUNK // 2, pair_body, (zero, zero))
    if NCHUNK % 2:
        acc, cnt = process(0, NCHUNK - 1, acc, cnt)

    stage_v[pl.ds(0, L)] = acc
    stage_v[pl.ds(L, L)] = cnt
    pltpu.sync_copy(stage_v, part_hbm.at[wid])


def _tc_body(regr_ref, gt_ref, mask_ref, loss_ref, cnt_ref):
    i = pl.program_id(0)
    m = mask_ref[...].astype(jnp.float32)
    d = jnp.abs(regr_ref[...] - gt_ref[...])
    me = jnp.broadcast_to(m[:, None, :], (PB, 2, B)).reshape(2 * PB, B)
    bl = jnp.sum(d * me, axis=0, keepdims=True)
    bc = jnp.sum(m, axis=0, keepdims=True)

    @pl.when(i == 0)
    def _():
        loss_ref[...] = jnp.zeros((1, B), jnp.float32)
        cnt_ref[...] = jnp.zeros((1, B), jnp.float32)

    loss_ref[...] += bl
    cnt_ref[...] += bc


def _combine_body(part_ref, tcl_ref, tcc_ref, out_ref):
    s_loss = jnp.sum(part_ref[:, :L]) + jnp.sum(tcl_ref[...])
    s_cnt = jnp.sum(part_ref[:, L:]) + jnp.sum(tcc_ref[...])
    num = s_cnt * 2.0
    out_ref[...] = jnp.full((1, L), s_loss / (num + 1e-4), jnp.float32)


@jax.jit
def _masked_l1(regr, gt_regr, mask):
    regr_f = jnp.transpose(regr, (1, 2, 0)).reshape(-1)
    gt_f = jnp.transpose(gt_regr, (1, 2, 0)).reshape(-1)
    mask_f = jnp.transpose(mask, (1, 0)).reshape(-1)
    regr_2 = regr_f.reshape(P * CHN, B)
    gt_2 = gt_f.reshape(P * CHN, B)
    mask_2 = mask_f.reshape(P, B)

    mesh = plsc.VectorSubcoreMesh(core_axis_name="c", subcore_axis_name="s")
    partials = pl.kernel(
        _partial_body,
        out_type=jax.ShapeDtypeStruct((NW, 2 * L), jnp.float32),
        mesh=mesh,
        compiler_params=pltpu.CompilerParams(
            needs_layout_passes=False, skip_device_barrier=True),
        scratch_types=[
            pltpu.VMEM((MSK_C,), jnp.int32),
            pltpu.VMEM((MSK_C,), jnp.int32),
            pltpu.VMEM((VAL_C,), jnp.float32),
            pltpu.VMEM((VAL_C,), jnp.float32),
            pltpu.VMEM((VAL_C,), jnp.float32),
            pltpu.VMEM((VAL_C,), jnp.float32),
            pltpu.VMEM((2 * L,), jnp.float32),
            pltpu.SemaphoreType.DMA,
            pltpu.SemaphoreType.DMA,
            pltpu.SemaphoreType.DMA,
            pltpu.SemaphoreType.DMA,
            pltpu.SemaphoreType.DMA,
            pltpu.SemaphoreType.DMA,
        ],
    )(regr_f, gt_f, mask_f)

    tc_loss, tc_cnt = pl.pallas_call(
        _tc_body,
        grid=(NTCB,),
        in_specs=[
            pl.BlockSpec((2 * PB, B), lambda i: (TCOFF + i, 0)),
            pl.BlockSpec((2 * PB, B), lambda i: (TCOFF + i, 0)),
            pl.BlockSpec((PB, B), lambda i: (TCOFF + i, 0)),
        ],
        out_specs=[
            pl.BlockSpec((1, B), lambda i: (0, 0)),
            pl.BlockSpec((1, B), lambda i: (0, 0)),
        ],
        out_shape=[
            jax.ShapeDtypeStruct((1, B), jnp.float32),
            jax.ShapeDtypeStruct((1, B), jnp.float32),
        ],
    )(regr_2, gt_2, mask_2)

    out = pl.pallas_call(
        _combine_body,
        out_shape=jax.ShapeDtypeStruct((1, L), jnp.float32),
    )(partials, tc_loss, tc_cnt)
    return out


def kernel(regr, gt_regr, mask):
    out = _masked_l1(regr, gt_regr, mask)
    return out[0, 0]


# speedup vs baseline: 1.0148x; 1.0136x over previous
import jax
import jax.numpy as jnp
from jax import lax
from jax.experimental import pallas as pl
from jax.experimental.pallas import tpu as pltpu
from jax.experimental.pallas import tpu_sc as plsc

NC = 2
NS = 16
NW = NC * NS
L = 16

B, P, CHN = 128, 20000, 2

P_SC = 12000
P_TC = P - P_SC
POS_W = P_SC // NW
CHUNK_P = 25
NCHUNK = POS_W // CHUNK_P
MSK_C = CHUNK_P * B
VAL_C = CHUNK_P * B * CHN

PB = 400
NTCB = P_TC // PB
TCOFF = P_SC // PB


def _partial_body(regr_hbm, gt_hbm, mask_hbm, part_hbm,
                  mk0, mk1, rr0, rr1, gg0, gg1, stage_v,
                  sm0, sm1, sr0, sr1, sg0, sg1):
    cid = lax.axis_index("c")
    sid = lax.axis_index("s")
    wid = cid * NS + sid

    mbase = wid * POS_W * B
    fbase = wid * POS_W * B * CHN

    mk = (mk0, mk1)
    rr = (rr0, rr1)
    gg = (gg0, gg1)
    sems = ((sm0, sr0, sg0), (sm1, sr1, sg1))

    def start(s, c):
        sm, sr, sg = sems[s]
        pltpu.async_copy(mask_hbm.at[pl.ds(mbase + c * MSK_C, MSK_C)], mk[s], sm)
        pltpu.async_copy(regr_hbm.at[pl.ds(fbase + c * VAL_C, VAL_C)], rr[s], sr)
        pltpu.async_copy(gt_hbm.at[pl.ds(fbase + c * VAL_C, VAL_C)], gg[s], sg)

    def wait(s):
        sm, sr, sg = sems[s]
        pltpu.make_async_copy(mask_hbm.at[pl.ds(0, MSK_C)], mk[s], sm).wait()
        pltpu.make_async_copy(regr_hbm.at[pl.ds(0, VAL_C)], rr[s], sr).wait()
        pltpu.make_async_copy(gt_hbm.at[pl.ds(0, VAL_C)], gg[s], sg).wait()

    def process(s, c, acc, cnt):
        wait(s)
        mk_s, rr_s, gg_s = mk[s], rr[s], gg[s]

        def pstep(p, carry, mk_s=mk_s, rr_s=rr_s, gg_s=gg_s):
            acc2, cnt2 = carry
            mo = p * B
            fo = p * B * CHN
            for j in range(B // L):
                m = mk_s[pl.ds(mo + j * L, L)].astype(jnp.float32)
                r0 = rr_s[pl.ds(fo + j * L, L)]
                g0 = gg_s[pl.ds(fo + j * L, L)]
                r1 = rr_s[pl.ds(fo + B + j * L, L)]
                g1 = gg_s[pl.ds(fo + B + j * L, L)]
                acc2 = acc2 + (jnp.abs(r0 - g0) + jnp.abs(r1 - g1)) * m
                cnt2 = cnt2 + m
            return acc2, cnt2

        acc, cnt = lax.fori_loop(0, CHUNK_P, pstep, (acc, cnt))

        @pl.when(c + 2 < NCHUNK)
        def _():
            start(s, c + 2)
        return acc, cnt

    zero = jnp.zeros((L,), jnp.float32)
    start(0, 0)
    start(1, 1)

    def pair_body(c2, carry):
        acc, cnt = carry
        acc, cnt = process(0, c2 * 2, acc, cnt)
        acc, cnt = process(1, c2 * 2 + 1, acc, cnt)
        return acc, cnt

    acc, cnt = lax.fori_loop(0, NCHUNK // 2, pair_body, (zero, zero))
    if NCHUNK % 2:
        acc, cnt = process(0, NCHUNK - 1, acc, cnt)

    stage_v[pl.ds(0, L)] = acc
    stage_v[pl.ds(L, L)] = cnt
    pltpu.sync_copy(stage_v, part_hbm.at[wid])


def _tc_body(regr_ref, gt_ref, mask_ref, loss_ref, cnt_ref):
    i = pl.program_id(0)
    m = mask_ref[...].astype(jnp.float32)
    d = jnp.abs(regr_ref[...] - gt_ref[...])
    me = jnp.broadcast_to(m[:, None, :], (PB, 2, B)).reshape(2 * PB, B)
    bl = jnp.sum(d * me, axis=0, keepdims=True)
    bc = jnp.sum(m, axis=0, keepdims=True)

    @pl.when(i == 0)
    def _():
        loss_ref[...] = jnp.zeros((1, B), jnp.float32)
        cnt_ref[...] = jnp.zeros((1, B), jnp.float32)

    loss_ref[...] += bl
    cnt_ref[...] += bc


def _combine_body(part_ref, tcl_ref, tcc_ref, out_ref):
    s_loss = jnp.sum(part_ref[:, :L]) + jnp.sum(tcl_ref[...])
    s_cnt = jnp.sum(part_ref[:, L:]) + jnp.sum(tcc_ref[...])
    num = s_cnt * 2.0
    out_ref[...] = jnp.full((1, L), s_loss / (num + 1e-4), jnp.float32)


@jax.jit
def _masked_l1(regr, gt_regr, mask):
    regr_f = jnp.transpose(regr, (1, 2, 0)).reshape(-1)
    gt_f = jnp.transpose(gt_regr, (1, 2, 0)).reshape(-1)
    mask_f = jnp.transpose(mask, (1, 0)).reshape(-1)
    regr_2 = regr_f.reshape(P * CHN, B)
    gt_2 = gt_f.reshape(P * CHN, B)
    mask_2 = mask_f.reshape(P, B)

    mesh = plsc.VectorSubcoreMesh(core_axis_name="c", subcore_axis_name="s")
    partials = pl.kernel(
        _partial_body,
        out_type=jax.ShapeDtypeStruct((NW, 2 * L), jnp.float32),
        mesh=mesh,
        compiler_params=pltpu.CompilerParams(needs_layout_passes=False),
        scratch_types=[
            pltpu.VMEM((MSK_C,), jnp.int32),
            pltpu.VMEM((MSK_C,), jnp.int32),
            pltpu.VMEM((VAL_C,), jnp.float32),
            pltpu.VMEM((VAL_C,), jnp.float32),
            pltpu.VMEM((VAL_C,), jnp.float32),
            pltpu.VMEM((VAL_C,), jnp.float32),
            pltpu.VMEM((2 * L,), jnp.float32),
            pltpu.SemaphoreType.DMA,
            pltpu.SemaphoreType.DMA,
            pltpu.SemaphoreType.DMA,
            pltpu.SemaphoreType.DMA,
            pltpu.SemaphoreType.DMA,
            pltpu.SemaphoreType.DMA,
        ],
    )(regr_f, gt_f, mask_f)

    tc_loss, tc_cnt = pl.pallas_call(
        _tc_body,
        grid=(NTCB,),
        in_specs=[
            pl.BlockSpec((2 * PB, B), lambda i: (TCOFF + i, 0)),
            pl.BlockSpec((2 * PB, B), lambda i: (TCOFF + i, 0)),
            pl.BlockSpec((PB, B), lambda i: (TCOFF + i, 0)),
        ],
        out_specs=[
            pl.BlockSpec((1, B), lambda i: (0, 0)),
            pl.BlockSpec((1, B), lambda i: (0, 0)),
        ],
        out_shape=[
            jax.ShapeDtypeStruct((1, B), jnp.float32),
            jax.ShapeDtypeStruct((1, B), jnp.float32),
        ],
    )(regr_2, gt_2, mask_2)

    out = pl.pallas_call(
        _combine_body,
        out_shape=jax.ShapeDtypeStruct((1, L), jnp.float32),
    )(partials, tc_loss, tc_cnt)
    return out


def kernel(regr, gt_regr, mask):
    out = _masked_l1(regr, gt_regr, mask)
    return out[0, 0]
